# baseline (speedup 1.0000x reference)
import functools

import jax
import jax.numpy as jnp
import numpy as np
from jax import lax
from jax.experimental import pallas as pl
from jax.experimental.pallas import tpu as pltpu
from jax.experimental.pallas import tpu_sc as plsc

DOUT = 64
BATCH = 16384
L = 50
LPAD = 56
VOCAB = 777
TROWS = 800
ZROW = 776

NC = 2
NS = 16
NW = NC * NS

BW = BATCH // NW
CHT = 2 * L
TPB = 2
CB = 2 * TPB
NCH = BW // CB
NTR = BW // 2
TOKW = BW * L


def _white_mask_np():
    m = np.zeros((1, 8, 8, 1), dtype=np.float32)
    for y in range(8):
        for x in range(8):
            m[0, y, x, 0] = float((y + x) % 2 == 0)
    return m


_MASK_NP = np.broadcast_to(_white_mask_np(), (1, 8, 8, DOUT)).copy()


def _table_body(tiles_ref, coord_ref, piece_ref, row_ref, col_ref, tc_ref,
                mask_ref, o_ref):
    o_ref[...] = (tiles_ref[...] + coord_ref[...] + piece_ref[...]
                  + row_ref[...] + col_ref[...]
                  + tc_ref[...] * mask_ref[...])


def _build_table(tiles, coord, piece, row, col, tilecolor):
    return pl.pallas_call(
        _table_body,
        out_shape=jax.ShapeDtypeStruct((12, 8, 8, DOUT), jnp.float32),
    )(tiles, coord, piece, row, col, tilecolor, jnp.asarray(_MASK_NP))


_PERM = np.empty((64,), dtype=np.int32)
for _j in range(16):
    _PERM[2 * _j] = _j
    _PERM[2 * _j + 1] = 16 + _j
    _PERM[32 + 2 * _j] = 32 + _j
    _PERM[32 + 2 * _j + 1] = 48 + _j

DW = DOUT // 2


def _sc_body(xp_hbm, table_hbm, bias_hbm, out_hbm,
             x_v, r0, r1, r2, r3, acc_v, bias_v, table_s,
             sem0, sem1, sem2, sem3):
    sid = lax.axis_index("s")
    wid = sid * NC + lax.axis_index("c")

    rows_per_tile = TROWS // NS
    pltpu.sync_copy(table_hbm.at[pl.ds(sid * rows_per_tile, rows_per_tile)],
                    table_s.at[pl.ds(sid * rows_per_tile, rows_per_tile)])
    plsc.subcore_barrier()

    pltpu.sync_copy(bias_hbm, bias_v)
    pltpu.sync_copy(xp_hbm.at[pl.ds(wid * NTR, NTR)], x_v)

    def fire(g, buf, sem):
        for h in range(TPB):
            idx = x_v.at[g * TPB + h]
            pltpu.async_copy(table_s.at[idx], buf.at[pl.ds(h * CHT, CHT)],
                             sem)

    def wait(buf, sem):
        for h in range(TPB):
            idx = x_v.at[0]
            pltpu.make_async_copy(table_s.at[idx],
                                  buf.at[pl.ds(h * CHT, CHT)], sem).wait()

    bufs = (r0, r1, r2, r3)
    sems = (sem0, sem1, sem2, sem3)
    for j in range(4):
        fire(j, bufs[j], sems[j])

    def unpack4(buf, t):
        a0, a1 = plsc.unpack(buf[t, pl.ds(0, 32)],
                             format=plsc.PackFormat.INTERLEAVED)
        a2, a3 = plsc.unpack(buf[t, pl.ds(32, 32)],
                             format=plsc.PackFormat.INTERLEAVED)
        return (a0, a1, a2, a3)

    def process(g, buf):
        for r in range(CB):
            c0 = unpack4(buf, r * L)
            c1 = unpack4(buf, r * L + 1)
            a = [[bias_v[pl.ds(k * 16, 16)] + c0[k], c1[k]] for k in range(4)]
            for l in range(2, L):
                c = unpack4(buf, r * L + l)
                for k in range(4):
                    a[k][l % 2] = a[k][l % 2] + c[k]
            lr = g * CB + r
            for k in range(4):
                acc_v[lr, pl.ds(k * 16, 16)] = a[k][0] + a[k][1]

    def outer(gg, carry):
        for j in range(4):
            g = gg * 4 + j
            wait(bufs[j], sems[j])
            process(g, bufs[j])

            @pl.when(g + 4 < NCH)
            def _():
                fire(g + 4, bufs[j], sems[j])

        return carry

    lax.fori_loop(0, NCH // 4, outer, 0)

    pltpu.sync_copy(acc_v, out_hbm.at[pl.ds(wid * BW, BW)])


_sc_emb = functools.partial(
    pl.kernel,
    out_type=jax.ShapeDtypeStruct((BATCH, DOUT), jnp.float32),
    mesh=plsc.VectorSubcoreMesh(core_axis_name="c", subcore_axis_name="s"),
    compiler_params=pltpu.CompilerParams(use_tc_tiling_on_sc=False,
                                         needs_layout_passes=False),
    scratch_types=[
        pltpu.VMEM((NTR, CHT), jnp.int32),
        pltpu.VMEM((TPB * CHT, DOUT), jnp.bfloat16),
        pltpu.VMEM((TPB * CHT, DOUT), jnp.bfloat16),
        pltpu.VMEM((TPB * CHT, DOUT), jnp.bfloat16),
        pltpu.VMEM((TPB * CHT, DOUT), jnp.bfloat16),
        pltpu.VMEM((BW, DOUT), jnp.float32),
        pltpu.VMEM((DOUT,), jnp.float32),
        pltpu.VMEM_SHARED((TROWS, DOUT), jnp.bfloat16),
        pltpu.SemaphoreType.DMA,
        pltpu.SemaphoreType.DMA,
        pltpu.SemaphoreType.DMA,
        pltpu.SemaphoreType.DMA,
    ],
)(_sc_body)


def kernel(x, misc, tiles, coord, piece, row, col, tilecolor, zeros, bias):
    w4 = _build_table(tiles, coord, piece, row, col, tilecolor)
    table = jnp.concatenate(
        [w4.reshape(768, DOUT), misc,
         jnp.zeros((TROWS - 768 - 8, DOUT), jnp.float32)], axis=0)
    tpacked = (table.reshape(TROWS, 2, 2, 16).transpose(0, 1, 3, 2)
               .reshape(TROWS, DOUT).astype(jnp.bfloat16))
    xp = x.reshape(BATCH // 2, CHT)
    return _sc_emb(xp, tpacked, bias)


# speedup vs baseline: 1.5777x; 1.5777x over previous
import functools

import jax
import jax.numpy as jnp
import numpy as np
from jax import lax
from jax.experimental import pallas as pl
from jax.experimental.pallas import tpu as pltpu
from jax.experimental.pallas import tpu_sc as plsc

DOUT = 64
BATCH = 16384
L = 50
LPAD = 56
VOCAB = 777
TROWS = 800
ZROW = 776

NC = 2
NS = 16
NW = NC * NS

BW = BATCH // NW
CHT = 2 * L
TPB = 1
CB = 2 * TPB
NCH = BW // CB
NTR = BW // 2
TOKW = BW * L


def _white_mask_np():
    m = np.zeros((1, 8, 8, 1), dtype=np.float32)
    for y in range(8):
        for x in range(8):
            m[0, y, x, 0] = float((y + x) % 2 == 0)
    return m


_MASK_NP = np.broadcast_to(_white_mask_np(), (1, 8, 8, DOUT)).copy()


def _table_body(tiles_ref, coord_ref, piece_ref, row_ref, col_ref, tc_ref,
                mask_ref, o_ref):
    o_ref[...] = (tiles_ref[...] + coord_ref[...] + piece_ref[...]
                  + row_ref[...] + col_ref[...]
                  + tc_ref[...] * mask_ref[...])


def _build_table(tiles, coord, piece, row, col, tilecolor):
    return pl.pallas_call(
        _table_body,
        out_shape=jax.ShapeDtypeStruct((12, 8, 8, DOUT), jnp.float32),
    )(tiles, coord, piece, row, col, tilecolor, jnp.asarray(_MASK_NP))


_PERM = np.empty((64,), dtype=np.int32)
for _j in range(16):
    _PERM[2 * _j] = _j
    _PERM[2 * _j + 1] = 16 + _j
    _PERM[32 + 2 * _j] = 32 + _j
    _PERM[32 + 2 * _j + 1] = 48 + _j

DW = DOUT // 2


def _sc_body(xp_hbm, table_hbm, bias_hbm, out_hbm,
             x_v, r0, r1, acc_v, bias_v, table_s, sem0, sem1):
    sid = lax.axis_index("s")
    wid = sid * NC + lax.axis_index("c")

    rows_per_tile = TROWS // NS
    pltpu.sync_copy(table_hbm.at[pl.ds(sid * rows_per_tile, rows_per_tile)],
                    table_s.at[pl.ds(sid * rows_per_tile, rows_per_tile)])
    plsc.subcore_barrier()

    pltpu.sync_copy(bias_hbm, bias_v)
    pltpu.sync_copy(xp_hbm.at[pl.ds(wid * NTR, NTR)], x_v)

    def fire(g, buf, sem):
        for h in range(TPB):
            idx = x_v.at[g * TPB + h]
            pltpu.async_copy(table_s.at[idx], buf.at[pl.ds(h * CHT, CHT)],
                             sem)

    def wait(buf, sem):
        for h in range(TPB):
            idx = x_v.at[0]
            pltpu.make_async_copy(table_s.at[idx],
                                  buf.at[pl.ds(h * CHT, CHT)], sem).wait()

    fire(0, r0, sem0)
    fire(1, r1, sem1)

    def unpack4(buf, t):
        a0, a1 = plsc.unpack(buf[t, pl.ds(0, 32)],
                             format=plsc.PackFormat.INTERLEAVED)
        a2, a3 = plsc.unpack(buf[t, pl.ds(32, 32)],
                             format=plsc.PackFormat.INTERLEAVED)
        return (a0, a1, a2, a3)

    def process(g, buf):
        for r in range(CB):
            c0 = unpack4(buf, r * L)
            c1 = unpack4(buf, r * L + 1)
            a = [[bias_v[pl.ds(k * 16, 16)] + c0[k], c1[k]] for k in range(4)]
            for l in range(2, L):
                c = unpack4(buf, r * L + l)
                for k in range(4):
                    a[k][l % 2] = a[k][l % 2] + c[k]
            lr = g * CB + r
            for k in range(4):
                acc_v[lr, pl.ds(k * 16, 16)] = a[k][0] + a[k][1]

    def outer(gg, carry):
        g0 = gg * 2
        g1 = gg * 2 + 1
        wait(r0, sem0)
        process(g0, r0)

        @pl.when(g0 + 2 < NCH)
        def _():
            fire(g0 + 2, r0, sem0)

        wait(r1, sem1)
        process(g1, r1)

        @pl.when(g1 + 2 < NCH)
        def _():
            fire(g1 + 2, r1, sem1)

        return carry

    lax.fori_loop(0, NCH // 2, outer, 0)

    pltpu.sync_copy(acc_v, out_hbm.at[pl.ds(wid * BW, BW)])


_sc_emb = functools.partial(
    pl.kernel,
    out_type=jax.ShapeDtypeStruct((BATCH, DOUT), jnp.float32),
    mesh=plsc.VectorSubcoreMesh(core_axis_name="c", subcore_axis_name="s"),
    compiler_params=pltpu.CompilerParams(use_tc_tiling_on_sc=False,
                                         needs_layout_passes=False),
    scratch_types=[
        pltpu.VMEM((NTR, CHT), jnp.int32),
        pltpu.VMEM((TPB * CHT, DOUT), jnp.bfloat16),
        pltpu.VMEM((TPB * CHT, DOUT), jnp.bfloat16),
        pltpu.VMEM((BW, DOUT), jnp.float32),
        pltpu.VMEM((DOUT,), jnp.float32),
        pltpu.VMEM_SHARED((TROWS, DOUT), jnp.bfloat16),
        pltpu.SemaphoreType.DMA,
        pltpu.SemaphoreType.DMA,
    ],
)(_sc_body)


def kernel(x, misc, tiles, coord, piece, row, col, tilecolor, zeros, bias):
    w4 = _build_table(tiles, coord, piece, row, col, tilecolor)
    table = jnp.concatenate(
        [w4.reshape(768, DOUT), misc,
         jnp.zeros((TROWS - 768 - 8, DOUT), jnp.float32)], axis=0)
    tpacked = (table.reshape(TROWS, 2, 2, 16).transpose(0, 1, 3, 2)
               .reshape(TROWS, DOUT).astype(jnp.bfloat16))
    xp = x.reshape(BATCH // 2, CHT)
    return _sc_emb(xp, tpacked, bias)


# speedup vs baseline: 1.6805x; 1.0652x over previous
import functools

import jax
import jax.numpy as jnp
import numpy as np
from jax import lax
from jax.experimental import pallas as pl
from jax.experimental.pallas import tpu as pltpu
from jax.experimental.pallas import tpu_sc as plsc

DOUT = 64
BATCH = 16384
L = 50
LPAD = 56
VOCAB = 777
TROWS = 800
ZROW = 776

NC = 2
NS = 16
NW = NC * NS

BW = BATCH // NW
CHT = 2 * L
TPB = 2
CB = 2 * TPB
NCH = BW // CB
NTR = BW // 2
TOKW = BW * L


def _white_mask_np():
    m = np.zeros((1, 8, 8, 1), dtype=np.float32)
    for y in range(8):
        for x in range(8):
            m[0, y, x, 0] = float((y + x) % 2 == 0)
    return m


_MASK_NP = np.broadcast_to(_white_mask_np(), (1, 8, 8, DOUT)).copy()


def _table_body(tiles_ref, coord_ref, piece_ref, row_ref, col_ref, tc_ref,
                mask_ref, o_ref):
    o_ref[...] = (tiles_ref[...] + coord_ref[...] + piece_ref[...]
                  + row_ref[...] + col_ref[...]
                  + tc_ref[...] * mask_ref[...])


def _build_table(tiles, coord, piece, row, col, tilecolor):
    return pl.pallas_call(
        _table_body,
        out_shape=jax.ShapeDtypeStruct((12, 8, 8, DOUT), jnp.float32),
    )(tiles, coord, piece, row, col, tilecolor, jnp.asarray(_MASK_NP))


_PERM = np.empty((64,), dtype=np.int32)
for _j in range(16):
    _PERM[2 * _j] = _j
    _PERM[2 * _j + 1] = 16 + _j
    _PERM[32 + 2 * _j] = 32 + _j
    _PERM[32 + 2 * _j + 1] = 48 + _j

DW = DOUT // 2


def _sc_body(xp_hbm, table_hbm, bias_hbm, out_hbm,
             x_v, r0, r1, acc_v, bias_v, table_s, sem0, sem1, osem):
    sid = lax.axis_index("s")
    wid = sid * NC + lax.axis_index("c")

    xcopy = pltpu.make_async_copy(xp_hbm.at[pl.ds(wid * NTR, NTR)], x_v,
                                  osem)
    xcopy.start()

    rows_per_tile = TROWS // NS
    pltpu.sync_copy(table_hbm.at[pl.ds(sid * rows_per_tile, rows_per_tile)],
                    table_s.at[pl.ds(sid * rows_per_tile, rows_per_tile)])
    plsc.subcore_barrier()

    pltpu.sync_copy(bias_hbm, bias_v)
    xcopy.wait()

    def fire(g, buf, sem):
        for h in range(TPB):
            idx = x_v.at[g * TPB + h]
            pltpu.async_copy(table_s.at[idx], buf.at[pl.ds(h * CHT, CHT)],
                             sem)

    def wait(buf, sem):
        for h in range(TPB):
            idx = x_v.at[0]
            pltpu.make_async_copy(table_s.at[idx],
                                  buf.at[pl.ds(h * CHT, CHT)], sem).wait()

    fire(0, r0, sem0)
    fire(1, r1, sem1)

    def unpack4(buf, t):
        a0, a1 = plsc.unpack(buf[t, pl.ds(0, 32)],
                             format=plsc.PackFormat.INTERLEAVED)
        a2, a3 = plsc.unpack(buf[t, pl.ds(32, 32)],
                             format=plsc.PackFormat.INTERLEAVED)
        return (a0, a1, a2, a3)

    def process(g, buf):
        for r in range(CB):
            c0 = unpack4(buf, r * L)
            c1 = unpack4(buf, r * L + 1)
            a = [[bias_v[pl.ds(k * 16, 16)] + c0[k], c1[k]] for k in range(4)]
            for l in range(2, L):
                c = unpack4(buf, r * L + l)
                for k in range(4):
                    a[k][l % 2] = a[k][l % 2] + c[k]
            lr = g * CB + r
            for k in range(4):
                acc_v[lr, pl.ds(k * 16, 16)] = a[k][0] + a[k][1]
        pltpu.async_copy(acc_v.at[pl.ds(g * CB, CB)],
                         out_hbm.at[pl.ds(wid * BW + g * CB, CB)], osem)

    def outer(gg, carry):
        g0 = gg * 2
        g1 = gg * 2 + 1
        wait(r0, sem0)
        process(g0, r0)

        @pl.when(g0 + 2 < NCH)
        def _():
            fire(g0 + 2, r0, sem0)

        wait(r1, sem1)
        process(g1, r1)

        @pl.when(g1 + 2 < NCH)
        def _():
            fire(g1 + 2, r1, sem1)

        return carry

    lax.fori_loop(0, NCH // 2, outer, 0)

    def drain(g, carry):
        pltpu.make_async_copy(acc_v.at[pl.ds(0, CB)],
                              out_hbm.at[pl.ds(wid * BW, CB)], osem).wait()
        return carry

    lax.fori_loop(0, NCH, drain, 0)


_sc_emb = functools.partial(
    pl.kernel,
    out_type=jax.ShapeDtypeStruct((BATCH, DOUT), jnp.float32),
    mesh=plsc.VectorSubcoreMesh(core_axis_name="c", subcore_axis_name="s"),
    compiler_params=pltpu.CompilerParams(use_tc_tiling_on_sc=False,
                                         needs_layout_passes=False),
    scratch_types=[
        pltpu.VMEM((NTR, CHT), jnp.int32),
        pltpu.VMEM((TPB * CHT, DOUT), jnp.bfloat16),
        pltpu.VMEM((TPB * CHT, DOUT), jnp.bfloat16),
        pltpu.VMEM((BW, DOUT), jnp.float32),
        pltpu.VMEM((DOUT,), jnp.float32),
        pltpu.VMEM_SHARED((TROWS, DOUT), jnp.bfloat16),
        pltpu.SemaphoreType.DMA,
        pltpu.SemaphoreType.DMA,
        pltpu.SemaphoreType.DMA,
    ],
)(_sc_body)


def kernel(x, misc, tiles, coord, piece, row, col, tilecolor, zeros, bias):
    w4 = _build_table(tiles, coord, piece, row, col, tilecolor)
    table = jnp.concatenate(
        [w4.reshape(768, DOUT), misc,
         jnp.zeros((TROWS - 768 - 8, DOUT), jnp.float32)], axis=0)
    tpacked = (table.reshape(TROWS, 2, 2, 16).transpose(0, 1, 3, 2)
               .reshape(TROWS, DOUT).astype(jnp.bfloat16))
    xp = x.reshape(BATCH // 2, CHT)
    return _sc_emb(xp, tpacked, bias)
